# baseline (speedup 1.0000x reference)
import functools
import math

import jax
import jax.numpy as jnp
from jax import lax
from jax.experimental import pallas as pl
from jax.experimental.pallas import tpu as pltpu
from jax.experimental.pallas import tpu_sc as plsc

_LFREQ = 10000.0
_LANES = 16


def _sinusoidal_pe(length, dm):
    pos = jnp.arange(length, dtype=jnp.float32)[:, None]
    i = jnp.arange(0, dm, 2, dtype=jnp.float32)
    div = jnp.exp(-(jnp.log(_LFREQ)) * i / dm)
    angles = pos * div[None, :]
    pe = jnp.zeros((length, dm), dtype=jnp.float32)
    pe = pe.at[:, 0::2].set(jnp.sin(angles))
    pe = pe.at[:, 1::2].set(jnp.cos(angles))
    return pe


@functools.partial(jax.jit, static_argnames=("bsz", "dm", "length"))
def _embed_sc(idx, pe, table, bsz, dm, length):
    info = plsc.get_sparse_core_info()
    nc, ns = info.num_cores, info.num_subcores
    nw = nc * ns
    n_chunks = bsz // nw
    half = dm // 2
    vecs_per_half = half // _LANES
    scale = jnp.float32(math.sqrt(dm))

    mesh = plsc.VectorSubcoreMesh(core_axis_name="c", subcore_axis_name="s")

    @functools.partial(
        pl.kernel,
        out_type=jax.ShapeDtypeStruct((bsz, length, dm), jnp.float32),
        mesh=mesh,
        compiler_params=pltpu.CompilerParams(use_tc_tiling_on_sc=False),
        scratch_types=[
            pltpu.VMEM((length,), jnp.int32),
            pltpu.VMEM((length,), jnp.int32),
            pltpu.VMEM((length * dm,), jnp.float32),
            pltpu.VMEM((length, dm), jnp.float32),
            pltpu.VMEM((length, dm), jnp.float32),
            pltpu.VMEM((length, dm), jnp.float32),
        ]
        + [pltpu.SemaphoreType.DMA] * 3,
    )
    def body(idx_hbm, pe_hbm, table_hbm, out_hbm, ic0, ic1, pe_v,
             x0, x1, y0, g0, g1, o0):
        xs, gs = (x0, x1), (g0, g1)
        ics = (ic0, ic1)
        wid = lax.axis_index("s") * nc + lax.axis_index("c")
        base_b = wid * n_chunks
        pltpu.sync_copy(pe_hbm, pe_v)

        def gather(c, i):
            return pltpu.make_async_copy(
                table_hbm.at[ics[i]], xs[i], gs[i]
            )

        def load_idx(c, i):
            pltpu.sync_copy(idx_hbm.at[base_b + c], ics[i])

        def out_copy(c):
            return pltpu.make_async_copy(
                y0, out_hbm.at[base_b + c], o0
            )

        load_idx(0, 0)
        gather(0, 0).start()

        def step(c, i):
            @pl.when(c + 1 < n_chunks)
            def _():
                load_idx(c + 1, 1 - i)

            gather(c, i).wait()

            @pl.when(c + 1 < n_chunks)
            def _():
                gather(c + 1, 1 - i).start()

            @pl.when(c >= 1)
            def _():
                out_copy(c - 1).wait()

            x = xs[i]

            def row_body(r, _2):
                pes = [pe_v[pl.ds(r * dm + j * _LANES, _LANES)]
                       for j in range(2 * vecs_per_half)]
                for j in range(2 * vecs_per_half):
                    sl = pl.ds(j * _LANES, _LANES)
                    y0[r, sl] = x[r, sl] * scale + pes[j]
                return 0

            lax.fori_loop(0, length, row_body, 0)
            out_copy(c).start()

        def round_body(k, _):
            step(2 * k, 0)
            step(2 * k + 1, 1)
            return 0

        lax.fori_loop(0, n_chunks // 2, round_body, 0)
        out_copy(n_chunks - 1).wait()

    return body(idx, pe, table)


def kernel(text, embed_table):
    b, l = text.shape
    v, dm = embed_table.shape
    idx = text.astype(jnp.int32)
    pe = _sinusoidal_pe(l, dm).reshape(-1)
    return _embed_sc(idx, pe, embed_table, b, dm, l)


# speedup vs baseline: 3.6819x; 3.6819x over previous
import functools
import math

import jax
import jax.numpy as jnp
from jax import lax
from jax.experimental import pallas as pl
from jax.experimental.pallas import tpu as pltpu
from jax.experimental.pallas import tpu_sc as plsc

_LFREQ = 10000.0
_LANES = 16


def _sinusoidal_pe(length, dm):
    pos = jnp.arange(length, dtype=jnp.float32)[:, None]
    i = jnp.arange(0, dm, 2, dtype=jnp.float32)
    div = jnp.exp(-(jnp.log(_LFREQ)) * i / dm)
    angles = pos * div[None, :]
    pe = jnp.zeros((length, dm), dtype=jnp.float32)
    pe = pe.at[:, 0::2].set(jnp.sin(angles))
    pe = pe.at[:, 1::2].set(jnp.cos(angles))
    return pe


@functools.partial(jax.jit, static_argnames=("bsz", "dm", "length"))
def _embed_sc(idx_t, pe, table, bsz, dm, length):
    info = plsc.get_sparse_core_info()
    nc, ns = info.num_cores, info.num_subcores
    nw = nc * ns
    bw = bsz // nw
    vecs_per_row = dm // _LANES
    scale = jnp.float32(math.sqrt(dm))

    mesh = plsc.VectorSubcoreMesh(core_axis_name="c", subcore_axis_name="s")

    @functools.partial(
        pl.kernel,
        out_type=jax.ShapeDtypeStruct((length, bsz, dm), jnp.float32),
        mesh=mesh,
        scratch_types=[
            pltpu.VMEM((bw,), jnp.int32),
            pltpu.VMEM((bw,), jnp.int32),
            pltpu.VMEM((length, dm), jnp.float32),
            pltpu.VMEM((bw, dm), jnp.float32),
            pltpu.VMEM((bw, dm), jnp.float32),
            pltpu.VMEM((bw, dm), jnp.float32),
            pltpu.VMEM((bw, dm), jnp.float32),
        ]
        + [pltpu.SemaphoreType.DMA] * 6,
    )
    def body(idx_hbm, pe_hbm, table_hbm, out_hbm, ic0, ic1, pe_v,
             x0, x1, y0, y1, i0, i1, g0, g1, o0, o1):
        ics, xs, ys = (ic0, ic1), (x0, x1), (y0, y1)
        isems, gsems, osems = (i0, i1), (g0, g1), (o0, o1)
        wid = lax.axis_index("s") * nc + lax.axis_index("c")
        base_b = wid * bw
        pltpu.sync_copy(pe_hbm, pe_v)

        def idx_copy(l, i):
            return pltpu.make_async_copy(
                idx_hbm.at[l, pl.ds(base_b, bw)], ics[i], isems[i]
            )

        def gather(i):
            return pltpu.make_async_copy(
                table_hbm.at[ics[i]], xs[i], gsems[i]
            )

        def out_copy(l, i):
            return pltpu.make_async_copy(
                ys[i], out_hbm.at[l, pl.ds(base_b, bw)], osems[i]
            )

        idx_copy(0, 0).start()
        idx_copy(0, 0).wait()
        gather(0).start()
        idx_copy(1, 1).start()

        def step(l, i):
            gather(i).wait()

            @pl.when(l + 2 < length)
            def _():
                idx_copy(l + 2, i).start()

            @pl.when(l + 1 < length)
            def _():
                idx_copy(l + 1, 1 - i).wait()
                gather(1 - i).start()

            @pl.when(l >= 2)
            def _():
                out_copy(l - 2, i).wait()

            x, y = xs[i], ys[i]
            pes = [pe_v[l, pl.ds(j * _LANES, _LANES)]
                   for j in range(vecs_per_row)]

            def row_body(r, _2):
                for j in range(vecs_per_row):
                    sl = pl.ds(j * _LANES, _LANES)
                    y[r, sl] = x[r, sl] * scale + pes[j]
                return 0

            lax.fori_loop(0, bw, row_body, 0)
            out_copy(l, i).start()

        def round_body(k, _):
            step(2 * k, 0)
            step(2 * k + 1, 1)
            return 0

        lax.fori_loop(0, length // 2, round_body, 0)
        out_copy(length - 2, 0).wait()
        out_copy(length - 1, 1).wait()

    return body(idx_t, pe, table)


def kernel(text, embed_table):
    b, l = text.shape
    v, dm = embed_table.shape
    idx_t = text.T.astype(jnp.int32)
    pe = _sinusoidal_pe(l, dm)
    out_t = _embed_sc(idx_t, pe, embed_table, b, dm, l)
    return out_t.transpose(1, 0, 2)


# speedup vs baseline: 3.8103x; 1.0349x over previous
import functools
import math

import jax
import jax.numpy as jnp
from jax import lax
from jax.experimental import pallas as pl
from jax.experimental.pallas import tpu as pltpu
from jax.experimental.pallas import tpu_sc as plsc

_LFREQ = 10000.0
_LANES = 16


def _sinusoidal_pe(length, dm):
    pos = jnp.arange(length, dtype=jnp.float32)[:, None]
    i = jnp.arange(0, dm, 2, dtype=jnp.float32)
    div = jnp.exp(-(jnp.log(_LFREQ)) * i / dm)
    angles = pos * div[None, :]
    pe = jnp.zeros((length, dm), dtype=jnp.float32)
    pe = pe.at[:, 0::2].set(jnp.sin(angles))
    pe = pe.at[:, 1::2].set(jnp.cos(angles))
    return pe


@functools.partial(jax.jit, static_argnames=("bsz", "dm", "length"))
def _embed_sc(idx_t, pe, table, bsz, dm, length):
    info = plsc.get_sparse_core_info()
    nc, ns = info.num_cores, info.num_subcores
    nw = nc * ns
    bw = bsz // nw
    vecs_per_row = dm // _LANES
    scale = jnp.float32(math.sqrt(dm))

    mesh = plsc.VectorSubcoreMesh(core_axis_name="c", subcore_axis_name="s")

    @functools.partial(
        pl.kernel,
        out_type=jax.ShapeDtypeStruct((length, bsz, dm), jnp.float32),
        mesh=mesh,
        scratch_types=[
            pltpu.VMEM((length * bw,), jnp.int32),
            pltpu.VMEM((length, dm), jnp.float32),
            pltpu.VMEM((bw, dm), jnp.float32),
            pltpu.VMEM((bw, dm), jnp.float32),
            pltpu.VMEM((bw, dm), jnp.float32),
            pltpu.VMEM((bw, dm), jnp.float32),
        ]
        + [pltpu.SemaphoreType.DMA] * 4,
    )
    def body(idx_hbm, pe_hbm, table_hbm, out_hbm, idx_v, pe_v,
             x0, x1, y0, y1, g0, g1, o0, o1):
        xs, ys = (x0, x1), (y0, y1)
        gsems, osems = (g0, g1), (o0, o1)
        wid = lax.axis_index("s") * nc + lax.axis_index("c")
        base_b = wid * bw
        pltpu.sync_copy(pe_hbm, pe_v)
        pltpu.sync_copy(idx_hbm.at[wid], idx_v)

        def gather(l, i):
            return pltpu.make_async_copy(
                table_hbm.at[idx_v.at[pl.ds(l * bw, bw)]], xs[i], gsems[i]
            )

        def out_copy(l, i):
            return pltpu.make_async_copy(
                ys[i], out_hbm.at[l, pl.ds(base_b, bw)], osems[i]
            )

        gather(0, 0).start()

        def step(l, i):
            gather(l, i).wait()

            @pl.when(l + 1 < length)
            def _():
                gather(l + 1, 1 - i).start()

            @pl.when(l >= 2)
            def _():
                out_copy(l - 2, i).wait()

            x, y = xs[i], ys[i]
            pes = [pe_v[l, pl.ds(j * _LANES, _LANES)]
                   for j in range(vecs_per_row)]

            def row_body(r, _2):
                for j in range(vecs_per_row):
                    sl = pl.ds(j * _LANES, _LANES)
                    y[r, sl] = x[r, sl] * scale + pes[j]
                return 0

            lax.fori_loop(0, bw, row_body, 0)
            out_copy(l, i).start()

        def round_body(k, _):
            step(2 * k, 0)
            step(2 * k + 1, 1)
            return 0

        lax.fori_loop(0, length // 2, round_body, 0)
        out_copy(length - 2, 0).wait()
        out_copy(length - 1, 1).wait()

    return body(idx_t, pe, table)


def kernel(text, embed_table):
    b, l = text.shape
    v, dm = embed_table.shape
    nw = 32
    bw = b // nw
    idx_perm = (
        text.astype(jnp.int32).T.reshape(l, nw, bw)
        .transpose(1, 0, 2).reshape(nw, l * bw)
    )
    pe = _sinusoidal_pe(l, dm)
    out_t = _embed_sc(idx_perm, pe, embed_table, b, dm, l)
    return out_t.transpose(1, 0, 2)


# speedup vs baseline: 4.3575x; 1.1436x over previous
import functools
import math

import jax
import jax.numpy as jnp
from jax import lax
from jax.experimental import pallas as pl
from jax.experimental.pallas import tpu as pltpu
from jax.experimental.pallas import tpu_sc as plsc

_LFREQ = 10000.0
_LANES = 16


def _sinusoidal_pe(length, dm):
    pos = jnp.arange(length, dtype=jnp.float32)[:, None]
    i = jnp.arange(0, dm, 2, dtype=jnp.float32)
    div = jnp.exp(-(jnp.log(_LFREQ)) * i / dm)
    angles = pos * div[None, :]
    pe = jnp.zeros((length, dm), dtype=jnp.float32)
    pe = pe.at[:, 0::2].set(jnp.sin(angles))
    pe = pe.at[:, 1::2].set(jnp.cos(angles))
    return pe


@functools.partial(jax.jit, static_argnames=("bsz", "dm", "length"))
def _embed_sc(idx_t, pe, table, bsz, dm, length):
    info = plsc.get_sparse_core_info()
    nc, ns = info.num_cores, info.num_subcores
    nw = nc * ns
    bw = bsz // nw
    vecs_per_row = dm // _LANES
    scale = jnp.float32(math.sqrt(dm))

    mesh = plsc.VectorSubcoreMesh(core_axis_name="c", subcore_axis_name="s")

    @functools.partial(
        pl.kernel,
        out_type=jax.ShapeDtypeStruct((length, bsz, dm), jnp.float32),
        mesh=mesh,
        scratch_types=[
            pltpu.VMEM((bw,), jnp.int32),
            pltpu.VMEM((bw,), jnp.int32),
            pltpu.VMEM((length, dm), jnp.float32),
            pltpu.VMEM((bw, dm), jnp.float32),
            pltpu.VMEM((bw, dm), jnp.float32),
            pltpu.VMEM((bw, dm), jnp.float32),
            pltpu.VMEM((bw, dm), jnp.float32),
        ]
        + [pltpu.SemaphoreType.DMA] * 6,
    )
    def body(idx_hbm, pe_hbm, table_hbm, out_hbm, ic0, ic1, pe_v,
             x0, x1, y0, y1, i0, i1, g0, g1, o0, o1):
        ics, xs, ys = (ic0, ic1), (x0, x1), (y0, y1)
        isems, gsems, osems = (i0, i1), (g0, g1), (o0, o1)
        wid = lax.axis_index("s") * nc + lax.axis_index("c")
        base_b = wid * bw
        pltpu.sync_copy(pe_hbm, pe_v)

        def idx_copy(l, i):
            return pltpu.make_async_copy(
                idx_hbm.at[l, pl.ds(base_b, bw)], ics[i], isems[i]
            )

        def gather(i):
            return pltpu.make_async_copy(
                table_hbm.at[ics[i]], xs[i], gsems[i]
            )

        def out_copy(l, i):
            return pltpu.make_async_copy(
                ys[i], out_hbm.at[l, pl.ds(base_b, bw)], osems[i]
            )

        idx_copy(0, 0).start()
        idx_copy(0, 0).wait()
        gather(0).start()
        idx_copy(1, 1).start()

        def step(l, i):
            @pl.when(l + 1 < length)
            def _():
                idx_copy(l + 1, 1 - i).wait()
                gather(1 - i).start()

            gather(i).wait()

            @pl.when(l + 2 < length)
            def _():
                idx_copy(l + 2, i).start()

            @pl.when(l >= 2)
            def _():
                out_copy(l - 2, i).wait()

            x, y = xs[i], ys[i]
            pes = [pe_v[l, pl.ds(j * _LANES, _LANES)]
                   for j in range(vecs_per_row)]

            def row_body(r, _2):
                for j in range(vecs_per_row):
                    sl = pl.ds(j * _LANES, _LANES)
                    y[r, sl] = x[r, sl] * scale + pes[j]
                return 0

            lax.fori_loop(0, bw, row_body, 0)
            out_copy(l, i).start()

        def round_body(k, _):
            step(2 * k, 0)
            step(2 * k + 1, 1)
            return 0

        lax.fori_loop(0, length // 2, round_body, 0)
        out_copy(length - 2, 0).wait()
        out_copy(length - 1, 1).wait()

    return body(idx_t, pe, table)


def kernel(text, embed_table):
    b, l = text.shape
    v, dm = embed_table.shape
    idx_t = text.T.astype(jnp.int32)
    pe = _sinusoidal_pe(l, dm)
    out_t = _embed_sc(idx_t, pe, embed_table, b, dm, l)
    return out_t.transpose(1, 0, 2)


# speedup vs baseline: 4.3858x; 1.0065x over previous
import functools
import math

import jax
import jax.numpy as jnp
from jax import lax
from jax.experimental import pallas as pl
from jax.experimental.pallas import tpu as pltpu
from jax.experimental.pallas import tpu_sc as plsc

_LFREQ = 10000.0
_LANES = 16


def _sinusoidal_pe(length, dm):
    pos = jnp.arange(length, dtype=jnp.float32)[:, None]
    i = jnp.arange(0, dm, 2, dtype=jnp.float32)
    div = jnp.exp(-(jnp.log(_LFREQ)) * i / dm)
    angles = pos * div[None, :]
    pe = jnp.zeros((length, dm), dtype=jnp.float32)
    pe = pe.at[:, 0::2].set(jnp.sin(angles))
    pe = pe.at[:, 1::2].set(jnp.cos(angles))
    return pe


@functools.partial(jax.jit, static_argnames=("bsz", "dm", "length"))
def _embed_sc(idx_t, pe, table, bsz, dm, length):
    info = plsc.get_sparse_core_info()
    nc, ns = info.num_cores, info.num_subcores
    nw = nc * ns
    bw = bsz // nw
    vecs_per_row = dm // _LANES
    scale = jnp.float32(math.sqrt(dm))

    mesh = plsc.VectorSubcoreMesh(core_axis_name="c", subcore_axis_name="s")

    @functools.partial(
        pl.kernel,
        out_type=jax.ShapeDtypeStruct((length, bsz, dm), jnp.float32),
        mesh=mesh,
        scratch_types=[
            pltpu.VMEM((bw,), jnp.int32),
            pltpu.VMEM((bw,), jnp.int32),
            pltpu.VMEM((length, dm), jnp.float32),
            pltpu.VMEM((bw, dm), jnp.float32),
            pltpu.VMEM((bw, dm), jnp.float32),
            pltpu.VMEM((bw, dm), jnp.float32),
            pltpu.VMEM((bw, dm), jnp.float32),
        ]
        + [pltpu.SemaphoreType.DMA] * 7,
    )
    def body(idx_hbm, pe_hbm, table_hbm, out_hbm, ic0, ic1, pe_v,
             x0, x1, y0, y1, i0, i1, g0, g1, o0, o1, psem):
        ics, xs, ys = (ic0, ic1), (x0, x1), (y0, y1)
        isems, gsems, osems = (i0, i1), (g0, g1), (o0, o1)
        wid = lax.axis_index("s") * nc + lax.axis_index("c")
        base_b = wid * bw
        pe_copy = pltpu.make_async_copy(pe_hbm, pe_v, psem)
        pe_copy.start()

        def idx_copy(l, i):
            return pltpu.make_async_copy(
                idx_hbm.at[l, pl.ds(base_b, bw)], ics[i], isems[i]
            )

        def gather(i):
            return pltpu.make_async_copy(
                table_hbm.at[ics[i]], xs[i], gsems[i]
            )

        def out_copy(l, i):
            return pltpu.make_async_copy(
                ys[i], out_hbm.at[l, pl.ds(base_b, bw)], osems[i]
            )

        idx_copy(0, 0).start()
        idx_copy(0, 0).wait()
        gather(0).start()
        idx_copy(1, 1).start()
        pe_copy.wait()

        def step(l, i):
            @pl.when(l + 1 < length)
            def _():
                idx_copy(l + 1, 1 - i).wait()
                gather(1 - i).start()

            gather(i).wait()

            @pl.when(l + 2 < length)
            def _():
                idx_copy(l + 2, i).start()

            @pl.when(l >= 2)
            def _():
                out_copy(l - 2, i).wait()

            x, y = xs[i], ys[i]
            pes = [pe_v[l, pl.ds(j * _LANES, _LANES)]
                   for j in range(vecs_per_row)]

            def row_body(r, _2):
                for j in range(vecs_per_row):
                    sl = pl.ds(j * _LANES, _LANES)
                    y[r, sl] = x[r, sl] * scale + pes[j]
                return 0

            lax.fori_loop(0, bw, row_body, 0)
            out_copy(l, i).start()

        def round_body(k, _):
            step(2 * k, 0)
            step(2 * k + 1, 1)
            return 0

        lax.fori_loop(0, length // 2, round_body, 0)
        out_copy(length - 2, 0).wait()
        out_copy(length - 1, 1).wait()

    return body(idx_t, pe, table)


def kernel(text, embed_table):
    b, l = text.shape
    v, dm = embed_table.shape
    idx_t = text.T.astype(jnp.int32)
    pe = _sinusoidal_pe(l, dm)
    out_t = _embed_sc(idx_t, pe, embed_table, b, dm, l)
    return out_t.transpose(1, 0, 2)
